# baseline (speedup 1.0000x reference)
import jax
import jax.numpy as jnp
from jax import lax
from jax.experimental import pallas as pl
from jax.experimental.pallas import tpu as pltpu

_MIB = 2 ** 20
_GROUP_RATIOS = (1, 1, 2, 4)


def _stats_tile_n(n_tokens, per_lane_bytes, budget_bytes=12 * _MIB, max_tn=4096):
    if n_tokens % 128 != 0 or n_tokens <= 128:
        return n_tokens
    cands = [t for t in range(128, min(n_tokens, max_tn) + 1, 128)
             if n_tokens % t == 0]
    if not cands:
        return n_tokens
    fitting = [t for t in cands if t * per_lane_bytes <= budget_bytes]
    return fitting[-1] if fitting else cands[0]


def _perm_from_stats(g, srow, inv_n, eps=1e-12):
    cross = lax.dot_general(
        srow, srow, dimension_numbers=(((1,), (1,)), ((), ())),
        preferred_element_type=jnp.float32)
    cov = g - cross * inv_n
    c = cov.shape[0]
    row = lax.broadcasted_iota(jnp.int32, (c, c), 0)
    col = lax.broadcasted_iota(jnp.int32, (c, c), 1)
    diag = jnp.where(row == col, cov, 0.0)
    var_col = jnp.maximum(jnp.sum(diag, axis=1, keepdims=True), eps)
    var_row = jnp.maximum(jnp.sum(diag, axis=0, keepdims=True), eps)
    corr = jnp.clip(cov * lax.rsqrt(var_col) * lax.rsqrt(var_row),
                    -1.0, 1.0)
    sim = jnp.mean(corr, axis=0, keepdims=True)
    sim_t = jnp.transpose(sim)
    gt = (sim_t > sim).astype(jnp.int32)
    eq_lt = ((sim_t == sim) & (row < col)).astype(jnp.int32)
    rank = jnp.sum(gt + eq_lt, axis=0, keepdims=True)
    return (row == rank).astype(jnp.float32)


def _make_mega_kernel(B, C, N, tn_dot, active):
    inv_b = 1.0 / float(B)
    inv_n = 1.0 / float(N)
    n_dot = N // tn_dot
    n_act = len(active)

    def body(q_hbm, k_hbm, v_hbm, *rest):
        outs = rest[:3 * n_act]
        qbuf, kbuf, vbuf, obuf, qsem, ksem, vsem, wsem = rest[3 * n_act:]

        qcp = pltpu.make_async_copy(q_hbm, qbuf, qsem)
        qcp.start()

        kcps, vcps, wcps = {}, {}, {}
        kv_slots = kbuf.shape[0]

        def start_kv(b):
            s = b % kv_slots
            kcps[b] = pltpu.make_async_copy(k_hbm.at[b], kbuf.at[s],
                                            ksem.at[s])
            vcps[b] = pltpu.make_async_copy(v_hbm.at[b], vbuf.at[s],
                                            vsem.at[s])
            kcps[b].start()
            vcps[b].start()

        for b in range(min(kv_slots, B)):
            start_kv(b)
        qcp.wait()

        g = None
        srow = None
        for t in range(n_dot):
            qt = qbuf[:, :, t * tn_dot:(t + 1) * tn_dot]
            s_t = jnp.sum(qt.astype(jnp.float32), axis=0) * inv_b
            d = lax.dot_general(
                s_t, s_t, dimension_numbers=(((1,), (1,)), ((), ())),
                preferred_element_type=jnp.float32)
            rs = jnp.sum(s_t, axis=1, keepdims=True)
            g = d if g is None else g + d
            srow = rs if srow is None else srow + rs
        pmat = _perm_from_stats(g, srow, inv_n)

        for b in range(B):
            slot = b % 2
            kv_slot = b % kv_slots
            if b >= 2:
                for cp in wcps[b - 2]:
                    cp.wait()
            kcps[b].wait()
            vcps[b].wait()
            srcs = (qbuf[b], kbuf[kv_slot], vbuf[kv_slot])
            for t in range(3):
                perm = lax.dot_general(
                    pmat.astype(srcs[t].dtype), srcs[t],
                    dimension_numbers=(((1,), (0,)), ((), ())),
                    preferred_element_type=jnp.float32)
                obuf[slot, t] = perm.astype(obuf.dtype)
            if b + kv_slots < B:
                start_kv(b + kv_slots)
            cps = []
            for t in range(3):
                for gi, (st, sz) in enumerate(active):
                    cp = pltpu.make_async_copy(
                        obuf.at[slot, t, pl.ds(st, sz)],
                        outs[t * n_act + gi].at[b],
                        wsem.at[slot])
                    cp.start()
                    cps.append(cp)
            wcps[b] = cps

        for b in (B - 2, B - 1):
            if 0 <= b < B:
                for cp in wcps[b]:
                    cp.wait()

    return body


def kernel(query, key, value):
    B, C, N = query.shape
    dtype = query.dtype
    itemsize = dtype.itemsize
    per_lane = 2 * B * C * itemsize + C * 4
    tn_dot = _stats_tile_n(N, per_lane)

    total = sum(_GROUP_RATIOS)
    sizes = [int(r / total * C) for r in _GROUP_RATIOS]
    starts, s = [], 0
    for sz in sizes:
        starts.append(s)
        s += sz
    active = [(st, sz) for st, sz in zip(starts, sizes) if sz > 0]
    n_act = len(active)

    q_act, k_act, v_act = [], [], []
    if active:
        any_spec = pl.BlockSpec(memory_space=pl.ANY)
        out_shape = (
            [jax.ShapeDtypeStruct((B, sz, N), query.dtype) for (_, sz) in active]
            + [jax.ShapeDtypeStruct((B, sz, N), key.dtype) for (_, sz) in active]
            + [jax.ShapeDtypeStruct((B, sz, N), value.dtype) for (_, sz) in active])

        outs = pl.pallas_call(
            _make_mega_kernel(B, C, N, tn_dot, active),
            out_shape=out_shape,
            in_specs=[any_spec, any_spec, any_spec],
            out_specs=[any_spec] * (3 * n_act),
            scratch_shapes=[
                pltpu.VMEM((B, C, N), dtype),
                pltpu.VMEM((3, C, N), dtype),
                pltpu.VMEM((3, C, N), dtype),
                pltpu.VMEM((2, 3, C, N), dtype),
                pltpu.SemaphoreType.DMA,
                pltpu.SemaphoreType.DMA((3,)),
                pltpu.SemaphoreType.DMA((3,)),
                pltpu.SemaphoreType.DMA((2,)),
            ],
            compiler_params=pltpu.CompilerParams(
                vmem_limit_bytes=56 * _MIB),
        )(query, key, value)
        q_act = list(outs[:n_act])
        k_act = list(outs[n_act:2 * n_act])
        v_act = list(outs[2 * n_act:3 * n_act])

    q_groups, k_groups, v_groups = [], [], []
    ai = 0
    for sz in sizes:
        if sz == 0:
            q_groups.append(jnp.zeros((B, 0, N), query.dtype))
            k_groups.append(jnp.zeros((B, 0, N), key.dtype))
            v_groups.append(jnp.zeros((B, 0, N), value.dtype))
        else:
            q_groups.append(q_act[ai])
            k_groups.append(k_act[ai])
            v_groups.append(v_act[ai])
            ai += 1
    return q_groups, k_groups, v_groups


# speedup vs baseline: 1.0075x; 1.0075x over previous
import jax
import jax.numpy as jnp
from jax import lax
from jax.experimental import pallas as pl
from jax.experimental.pallas import tpu as pltpu

_MIB = 2 ** 20
_GROUP_RATIOS = (1, 1, 2, 4)


def _stats_tile_n(n_tokens, per_lane_bytes, budget_bytes=12 * _MIB, max_tn=4096):
    if n_tokens % 128 != 0 or n_tokens <= 128:
        return n_tokens
    cands = [t for t in range(128, min(n_tokens, max_tn) + 1, 128)
             if n_tokens % t == 0]
    if not cands:
        return n_tokens
    fitting = [t for t in cands if t * per_lane_bytes <= budget_bytes]
    return fitting[-1] if fitting else cands[0]


def _perm_from_stats(g, srow, inv_n, eps=1e-12):
    cross = lax.dot_general(
        srow, srow, dimension_numbers=(((1,), (1,)), ((), ())),
        preferred_element_type=jnp.float32)
    cov = g - cross * inv_n
    c = cov.shape[0]
    row = lax.broadcasted_iota(jnp.int32, (c, c), 0)
    col = lax.broadcasted_iota(jnp.int32, (c, c), 1)
    diag = jnp.where(row == col, cov, 0.0)
    var_col = jnp.maximum(jnp.sum(diag, axis=1, keepdims=True), eps)
    var_row = jnp.maximum(jnp.sum(diag, axis=0, keepdims=True), eps)
    corr = jnp.clip(cov * lax.rsqrt(var_col) * lax.rsqrt(var_row),
                    -1.0, 1.0)
    sim = jnp.mean(corr, axis=0, keepdims=True)
    sim_t = jnp.transpose(sim)
    gt = (sim_t > sim).astype(jnp.int32)
    eq_lt = ((sim_t == sim) & (row < col)).astype(jnp.int32)
    rank = jnp.sum(gt + eq_lt, axis=0, keepdims=True)
    return (row == rank).astype(jnp.float32)


def _make_mega_kernel(B, C, N, tn_dot, active):
    inv_b = 1.0 / float(B)
    inv_n = 1.0 / float(N)
    n_dot = N // tn_dot
    n_act = len(active)

    def body(q_hbm, k_hbm, v_hbm, *rest):
        outs = rest[:3 * n_act]
        qbuf, kbuf, vbuf, obuf, qsem, ksem, vsem, wsem = rest[3 * n_act:]

        qcp = pltpu.make_async_copy(q_hbm, qbuf, qsem)
        qcp.start()

        kcps, vcps, wcps = {}, {}, {}
        kv_slots = kbuf.shape[0]

        def start_kv(b):
            s = b % kv_slots
            kcps[b] = pltpu.make_async_copy(k_hbm.at[b], kbuf.at[s],
                                            ksem.at[s])
            vcps[b] = pltpu.make_async_copy(v_hbm.at[b], vbuf.at[s],
                                            vsem.at[s])
            kcps[b].start()
            vcps[b].start()

        for b in range(min(kv_slots, B)):
            start_kv(b)
        qcp.wait()

        g = None
        srow = None
        for t in range(n_dot):
            qt = qbuf[:, :, t * tn_dot:(t + 1) * tn_dot]
            s_t = jnp.sum(qt.astype(jnp.float32), axis=0) * inv_b
            d = lax.dot_general(
                s_t, s_t, dimension_numbers=(((1,), (1,)), ((), ())),
                preferred_element_type=jnp.float32)
            rs = jnp.sum(s_t, axis=1, keepdims=True)
            g = d if g is None else g + d
            srow = rs if srow is None else srow + rs
        pmat = _perm_from_stats(g, srow, inv_n)

        for b in range(B):
            slot = b % 2
            kv_slot = b % kv_slots
            if b >= 2:
                for cp in wcps[b - 2]:
                    cp.wait()
            kcps[b].wait()
            vcps[b].wait()
            srcs = (qbuf[b], kbuf[kv_slot], vbuf[kv_slot])
            cps = []
            for t in range(3):
                perm = lax.dot_general(
                    pmat.astype(srcs[t].dtype), srcs[t],
                    dimension_numbers=(((1,), (0,)), ((), ())),
                    preferred_element_type=jnp.float32)
                obuf[slot, t] = perm.astype(obuf.dtype)
                for gi, (st, sz) in enumerate(active):
                    cp = pltpu.make_async_copy(
                        obuf.at[slot, t, pl.ds(st, sz)],
                        outs[t * n_act + gi].at[b],
                        wsem.at[slot])
                    cp.start()
                    cps.append(cp)
            wcps[b] = cps
            if b + kv_slots < B:
                start_kv(b + kv_slots)

        for b in (B - 2, B - 1):
            if 0 <= b < B:
                for cp in wcps[b]:
                    cp.wait()

    return body


def kernel(query, key, value):
    B, C, N = query.shape
    dtype = query.dtype
    itemsize = dtype.itemsize
    per_lane = 2 * B * C * itemsize + C * 4
    tn_dot = _stats_tile_n(N, per_lane)

    total = sum(_GROUP_RATIOS)
    sizes = [int(r / total * C) for r in _GROUP_RATIOS]
    starts, s = [], 0
    for sz in sizes:
        starts.append(s)
        s += sz
    active = [(st, sz) for st, sz in zip(starts, sizes) if sz > 0]
    n_act = len(active)

    q_act, k_act, v_act = [], [], []
    if active:
        any_spec = pl.BlockSpec(memory_space=pl.ANY)
        out_shape = (
            [jax.ShapeDtypeStruct((B, sz, N), query.dtype) for (_, sz) in active]
            + [jax.ShapeDtypeStruct((B, sz, N), key.dtype) for (_, sz) in active]
            + [jax.ShapeDtypeStruct((B, sz, N), value.dtype) for (_, sz) in active])

        outs = pl.pallas_call(
            _make_mega_kernel(B, C, N, tn_dot, active),
            out_shape=out_shape,
            in_specs=[any_spec, any_spec, any_spec],
            out_specs=[any_spec] * (3 * n_act),
            scratch_shapes=[
                pltpu.VMEM((B, C, N), dtype),
                pltpu.VMEM((3, C, N), dtype),
                pltpu.VMEM((3, C, N), dtype),
                pltpu.VMEM((2, 3, C, N), dtype),
                pltpu.SemaphoreType.DMA,
                pltpu.SemaphoreType.DMA((3,)),
                pltpu.SemaphoreType.DMA((3,)),
                pltpu.SemaphoreType.DMA((2,)),
            ],
            compiler_params=pltpu.CompilerParams(
                vmem_limit_bytes=56 * _MIB),
        )(query, key, value)
        q_act = list(outs[:n_act])
        k_act = list(outs[n_act:2 * n_act])
        v_act = list(outs[2 * n_act:3 * n_act])

    q_groups, k_groups, v_groups = [], [], []
    ai = 0
    for sz in sizes:
        if sz == 0:
            q_groups.append(jnp.zeros((B, 0, N), query.dtype))
            k_groups.append(jnp.zeros((B, 0, N), key.dtype))
            v_groups.append(jnp.zeros((B, 0, N), value.dtype))
        else:
            q_groups.append(q_act[ai])
            k_groups.append(k_act[ai])
            v_groups.append(v_act[ai])
            ai += 1
    return q_groups, k_groups, v_groups


# speedup vs baseline: 1.0479x; 1.0401x over previous
import jax
import jax.numpy as jnp
from jax import lax
from jax.experimental import pallas as pl
from jax.experimental.pallas import tpu as pltpu

_MIB = 2 ** 20
_GROUP_RATIOS = (1, 1, 2, 4)


def _stats_tile_n(n_tokens, per_lane_bytes, budget_bytes=12 * _MIB, max_tn=4096):
    if n_tokens % 128 != 0 or n_tokens <= 128:
        return n_tokens
    cands = [t for t in range(128, min(n_tokens, max_tn) + 1, 128)
             if n_tokens % t == 0]
    if not cands:
        return n_tokens
    fitting = [t for t in cands if t * per_lane_bytes <= budget_bytes]
    return fitting[-1] if fitting else cands[0]


def _perm_from_stats(g, srow, inv_n, eps=1e-12):
    cross = lax.dot_general(
        srow, srow, dimension_numbers=(((1,), (1,)), ((), ())),
        preferred_element_type=jnp.float32)
    cov = g - cross * inv_n
    c = cov.shape[0]
    row = lax.broadcasted_iota(jnp.int32, (c, c), 0)
    col = lax.broadcasted_iota(jnp.int32, (c, c), 1)
    diag = jnp.where(row == col, cov, 0.0)
    var_col = jnp.maximum(jnp.sum(diag, axis=1, keepdims=True), eps)
    var_row = jnp.maximum(jnp.sum(diag, axis=0, keepdims=True), eps)
    corr = jnp.clip(cov * lax.rsqrt(var_col) * lax.rsqrt(var_row),
                    -1.0, 1.0)
    sim = jnp.mean(corr, axis=0, keepdims=True)
    sim_t = jnp.transpose(sim)
    gt = (sim_t > sim).astype(jnp.int32)
    eq_lt = ((sim_t == sim) & (row < col)).astype(jnp.int32)
    rank = jnp.sum(gt + eq_lt, axis=0, keepdims=True)
    return (row == rank).astype(jnp.float32)


def _make_mega_kernel(B, C, N, tn_dot, active):
    inv_b = 1.0 / float(B)
    inv_n = 1.0 / float(N)
    n_dot = N // tn_dot
    n_act = len(active)

    def body(q_hbm, k_hbm, v_hbm, *rest):
        outs = rest[:3 * n_act]
        qbuf, kbuf, vbuf, obuf, qsem, ksem, vsem, wsem = rest[3 * n_act:]

        qcp = pltpu.make_async_copy(q_hbm, qbuf, qsem)
        qcp.start()

        kcps, vcps, wcps = {}, {}, {}
        kv_slots, grp = kbuf.shape[0], kbuf.shape[1]
        n_grp = B // grp

        def start_kv(p):
            s = p % kv_slots
            kcps[p] = pltpu.make_async_copy(k_hbm.at[pl.ds(p * grp, grp)],
                                            kbuf.at[s], ksem.at[s])
            vcps[p] = pltpu.make_async_copy(v_hbm.at[pl.ds(p * grp, grp)],
                                            vbuf.at[s], vsem.at[s])
            kcps[p].start()
            vcps[p].start()

        for p in range(min(kv_slots, n_grp)):
            start_kv(p)
        qcp.wait()

        g = None
        srow = None
        for t in range(n_dot):
            qt = qbuf[:, :, t * tn_dot:(t + 1) * tn_dot]
            s_t = jnp.sum(qt.astype(jnp.float32), axis=0) * inv_b
            d = lax.dot_general(
                s_t, s_t, dimension_numbers=(((1,), (1,)), ((), ())),
                preferred_element_type=jnp.float32)
            rs = jnp.sum(s_t, axis=1, keepdims=True)
            g = d if g is None else g + d
            srow = rs if srow is None else srow + rs
        pmat = _perm_from_stats(g, srow, inv_n)

        for b in range(B):
            slot = b % 2
            p, j = b // grp, b % grp
            kv_slot = p % kv_slots
            if b >= 2:
                for cp in wcps[b - 2]:
                    cp.wait()
            if j == 0:
                kcps[p].wait()
                vcps[p].wait()
            srcs = (qbuf[b], kbuf[kv_slot, j], vbuf[kv_slot, j])
            cps = []
            for t in range(3):
                perm = lax.dot_general(
                    pmat.astype(srcs[t].dtype), srcs[t],
                    dimension_numbers=(((1,), (0,)), ((), ())),
                    preferred_element_type=jnp.float32)
                obuf[slot, t] = perm.astype(obuf.dtype)
                for gi, (st, sz) in enumerate(active):
                    cp = pltpu.make_async_copy(
                        obuf.at[slot, t, pl.ds(st, sz)],
                        outs[t * n_act + gi].at[b],
                        wsem.at[slot])
                    cp.start()
                    cps.append(cp)
            wcps[b] = cps
            if j == grp - 1 and p + kv_slots < n_grp:
                start_kv(p + kv_slots)

        for b in (B - 2, B - 1):
            if 0 <= b < B:
                for cp in wcps[b]:
                    cp.wait()

    return body


def kernel(query, key, value):
    B, C, N = query.shape
    dtype = query.dtype
    itemsize = dtype.itemsize
    per_lane = 2 * B * C * itemsize + C * 4
    tn_dot = _stats_tile_n(N, per_lane)

    total = sum(_GROUP_RATIOS)
    sizes = [int(r / total * C) for r in _GROUP_RATIOS]
    starts, s = [], 0
    for sz in sizes:
        starts.append(s)
        s += sz
    active = [(st, sz) for st, sz in zip(starts, sizes) if sz > 0]
    n_act = len(active)

    q_act, k_act, v_act = [], [], []
    if active:
        kv_grp = 2 if B % 2 == 0 else 1
        any_spec = pl.BlockSpec(memory_space=pl.ANY)
        out_shape = (
            [jax.ShapeDtypeStruct((B, sz, N), query.dtype) for (_, sz) in active]
            + [jax.ShapeDtypeStruct((B, sz, N), key.dtype) for (_, sz) in active]
            + [jax.ShapeDtypeStruct((B, sz, N), value.dtype) for (_, sz) in active])

        outs = pl.pallas_call(
            _make_mega_kernel(B, C, N, tn_dot, active),
            out_shape=out_shape,
            in_specs=[any_spec, any_spec, any_spec],
            out_specs=[any_spec] * (3 * n_act),
            scratch_shapes=[
                pltpu.VMEM((B, C, N), dtype),
                pltpu.VMEM((2, kv_grp, C, N), dtype),
                pltpu.VMEM((2, kv_grp, C, N), dtype),
                pltpu.VMEM((2, 3, C, N), dtype),
                pltpu.SemaphoreType.DMA,
                pltpu.SemaphoreType.DMA((2,)),
                pltpu.SemaphoreType.DMA((2,)),
                pltpu.SemaphoreType.DMA((2,)),
            ],
            compiler_params=pltpu.CompilerParams(
                vmem_limit_bytes=56 * _MIB),
        )(query, key, value)
        q_act = list(outs[:n_act])
        k_act = list(outs[n_act:2 * n_act])
        v_act = list(outs[2 * n_act:3 * n_act])

    q_groups, k_groups, v_groups = [], [], []
    ai = 0
    for sz in sizes:
        if sz == 0:
            q_groups.append(jnp.zeros((B, 0, N), query.dtype))
            k_groups.append(jnp.zeros((B, 0, N), key.dtype))
            v_groups.append(jnp.zeros((B, 0, N), value.dtype))
        else:
            q_groups.append(q_act[ai])
            k_groups.append(k_act[ai])
            v_groups.append(v_act[ai])
            ai += 1
    return q_groups, k_groups, v_groups
